# Initial kernel scaffold; baseline (speedup 1.0000x reference)
import jax
import jax.numpy as jnp
from jax.experimental import pallas as pl


def kernel(x, edge_index, source_node, target_node, W1, b1, W2, b2, fc1_w, fc1_b, fc2_w, fc2_b):
    raise NotImplementedError("write your pallas kernel here")


# speedup vs baseline: 10.0319x; 10.0319x over previous
import functools

import jax
import jax.numpy as jnp
from jax import lax
from jax.experimental import pallas as pl
from jax.experimental.pallas import tpu as pltpu
from jax.experimental.pallas import tpu_sc as plsc

N, E, B, D, H1 = 10000, 100000, 8192, 768, 128
NC, NS, L = 2, 16, 16
N_PAD = 10240
NF = 6
F = D // NF
EB = 128
NBT = 56
NEB = NBT * NS
E_PAD = NEB * EB
RPT = N_PAD // NS
MB = 512

_MESH = plsc.VectorSubcoreMesh(
    core_axis_name="c", subcore_axis_name="s", num_cores=NC, num_subcores=NS)


def _deg_body(dst2d, deg_hbm, dst_v, ones_v, deg_v, hist_sh):
    cid = lax.axis_index("c")
    sid = lax.axis_index("s")
    for i in range(RPT // L):
        ones_v[pl.ds(L * i, L)] = jnp.full((L,), 1.0, jnp.float32)
    pltpu.sync_copy(ones_v, hist_sh.at[pl.ds(sid * RPT, RPT)])
    pltpu.sync_copy(dst2d.at[pl.ds(sid * NBT, NBT)], dst_v)
    plsc.subcore_barrier()

    @pl.loop(0, NBT)
    def _(b):
        pltpu.sync_copy(ones_v.at[pl.ds(0, EB)],
                        hist_sh.at[dst_v.at[b]], add=True)

    plsc.subcore_barrier()
    @pl.when(cid == 0)
    def _():
        pltpu.sync_copy(hist_sh.at[pl.ds(sid * RPT, RPT)], deg_v)
        pltpu.sync_copy(deg_v, deg_hbm.at[pl.ds(sid * RPT, RPT)])


def _sc_degree(dst2d):
    return pl.kernel(
        _deg_body,
        out_type=jax.ShapeDtypeStruct((N_PAD,), jnp.float32),
        mesh=_MESH,
        scratch_types=[
            pltpu.VMEM((NBT, EB), jnp.int32),
            pltpu.VMEM((RPT,), jnp.float32),
            pltpu.VMEM((RPT,), jnp.float32),
            pltpu.VMEM_SHARED((N_PAD,), jnp.float32),
        ],
    )(dst2d)


def _agg_body(xs_hbm, src2d, dst2d, out_hbm,
              src_v, dst_v, buf0, buf1, chunk, sem0, sem1):
    cid = lax.axis_index("c")
    sid = lax.axis_index("s")
    pltpu.sync_copy(src2d.at[pl.ds(sid * NBT, NBT)], src_v)
    pltpu.sync_copy(dst2d.at[pl.ds(sid * NBT, NBT)], dst_v)
    nsub = RPT // EB
    for kk in range(NF // NC):
        k = cid * (NF // NC) + kk
        for i in range(nsub):
            r0 = sid * RPT + i * EB
            pltpu.sync_copy(xs_hbm.at[k, pl.ds(r0, EB)], buf0)
            pltpu.sync_copy(buf0, chunk.at[pl.ds(r0, EB)])
        plsc.subcore_barrier()

        def _gath(i, buf, sem):
            return pltpu.async_copy(xs_hbm.at[k].at[src_v.at[i]], buf, sem)

        _gath(0, buf0, sem0)

        @pl.loop(0, NBT // 2)
        def _(g):
            i0 = 2 * g
            _gath(i0 + 1, buf1, sem1)
            pltpu.make_async_copy(
                xs_hbm.at[k].at[src_v.at[i0]], buf0, sem0).wait()
            pltpu.sync_copy(buf0, chunk.at[dst_v.at[i0]], add=True)

            @pl.when(g < NBT // 2 - 1)
            def _():
                _gath(i0 + 2, buf0, sem0)

            pltpu.make_async_copy(
                xs_hbm.at[k].at[src_v.at[i0 + 1]], buf1, sem1).wait()
            pltpu.sync_copy(buf1, chunk.at[dst_v.at[i0 + 1]], add=True)

        plsc.subcore_barrier()
        for i in range(nsub):
            r0 = sid * RPT + i * EB
            pltpu.sync_copy(chunk.at[pl.ds(r0, EB)], buf0)
            pltpu.sync_copy(buf0, out_hbm.at[k, pl.ds(r0, EB)])
        plsc.subcore_barrier()


def _sc_aggregate(xs, src2d, dst2d):
    return pl.kernel(
        _agg_body,
        out_type=jax.ShapeDtypeStruct((NF, N_PAD, F), jnp.float32),
        mesh=_MESH,
        scratch_types=[
            pltpu.VMEM((NBT, EB), jnp.int32),
            pltpu.VMEM((NBT, EB), jnp.int32),
            pltpu.VMEM((EB, F), jnp.float32),
            pltpu.VMEM((EB, F), jnp.float32),
            pltpu.VMEM_SHARED((N_PAD, F), jnp.float32),
            pltpu.SemaphoreType.DMA,
            pltpu.SemaphoreType.DMA,
        ],
    )(xs, src2d, dst2d)


def _link_body(s2, dinv_hbm, sn2d, tn2d, ss, st, dsv, dtv,
               idx_v, buf0, dval_v):
    cid = lax.axis_index("c")
    sid = lax.axis_index("s")
    w = cid * NS + sid
    bpw = B // (NC * NS)
    nb = bpw // EB
    for nodes, rows_out, dv_out in ((sn2d, ss, dsv), (tn2d, st, dtv)):
        pltpu.sync_copy(nodes, idx_v)
        for b in range(nb):
            r0 = w * bpw + b * EB
            row = w * nb + b
            pltpu.sync_copy(dinv_hbm.at[idx_v.at[row]], dval_v)
            pltpu.sync_copy(dval_v, dv_out.at[pl.ds(r0, EB)])
            for k in range(NF):
                pltpu.sync_copy(s2.at[k].at[idx_v.at[row]], buf0)
                pltpu.sync_copy(buf0, rows_out.at[k, pl.ds(r0, EB)])


def _sc_linkgather(s2, dinv, sn2d, tn2d):
    return pl.kernel(
        _link_body,
        out_type=(
            jax.ShapeDtypeStruct((NF, B, F), jnp.float32),
            jax.ShapeDtypeStruct((NF, B, F), jnp.float32),
            jax.ShapeDtypeStruct((B,), jnp.float32),
            jax.ShapeDtypeStruct((B,), jnp.float32),
        ),
        mesh=_MESH,
        scratch_types=[
            pltpu.VMEM((B // EB, EB), jnp.int32),
            pltpu.VMEM((EB, F), jnp.float32),
            pltpu.VMEM((EB,), jnp.float32),
        ],
    )(s2, dinv, sn2d, tn2d)


def _mm_scale_body(x_ref, w_ref, deg_ref, out_ref, dinv_ref):
    dinv = lax.rsqrt(deg_ref[...])
    dinv_ref[...] = dinv
    acc = jnp.dot(x_ref[...], w_ref[...], preferred_element_type=jnp.float32)
    acc = acc * dinv
    for k in range(NF):
        out_ref[k, :, :] = acc[:, k * F:(k + 1) * F]


def _tc_matmul_scale(x_pad, w, deg_col):
    return pl.pallas_call(
        _mm_scale_body,
        grid=(N_PAD // MB,),
        in_specs=[
            pl.BlockSpec((MB, D), lambda m: (m, 0)),
            pl.BlockSpec((D, D), lambda m: (0, 0)),
            pl.BlockSpec((MB, 1), lambda m: (m, 0)),
        ],
        out_specs=[
            pl.BlockSpec((NF, MB, F), lambda m: (0, m, 0)),
            pl.BlockSpec((MB, 1), lambda m: (m, 0)),
        ],
        out_shape=[
            jax.ShapeDtypeStruct((NF, N_PAD, F), jnp.float32),
            jax.ShapeDtypeStruct((N_PAD, 1), jnp.float32),
        ],
    )(x_pad, w, deg_col)


def _layer2_body(s_ref, dinv_ref, b1_ref, w_ref, out_ref):
    s = jnp.concatenate([s_ref[k] for k in range(NF)], axis=1)
    h = jax.nn.relu(dinv_ref[...] * s + b1_ref[...])
    acc = jnp.dot(h, w_ref[...], preferred_element_type=jnp.float32)
    acc = acc * dinv_ref[...]
    for k in range(NF):
        out_ref[k, :, :] = acc[:, k * F:(k + 1) * F]


def _tc_layer2(s1, dinv_col, b1_row, w2):
    return pl.pallas_call(
        _layer2_body,
        grid=(N_PAD // MB,),
        in_specs=[
            pl.BlockSpec((NF, MB, F), lambda m: (0, m, 0)),
            pl.BlockSpec((MB, 1), lambda m: (m, 0)),
            pl.BlockSpec((1, D), lambda m: (0, 0)),
            pl.BlockSpec((D, D), lambda m: (0, 0)),
        ],
        out_specs=pl.BlockSpec((NF, MB, F), lambda m: (0, m, 0)),
        out_shape=jax.ShapeDtypeStruct((NF, N_PAD, F), jnp.float32),
    )(s1, dinv_col, b1_row, w2)


def _head_body(ss_ref, st_ref, ds_ref, dt_ref, b2_ref, fc1w_ref, fc1b_ref,
               fc2w_ref, fc2b_ref, out_ref):
    sraw = jnp.concatenate([ss_ref[k] for k in range(NF)], axis=1)
    traw = jnp.concatenate([st_ref[k] for k in range(NF)], axis=1)
    hs = ds_ref[...] * sraw + b2_ref[...]
    ht = dt_ref[...] * traw + b2_ref[...]
    z = hs * ht
    t = jax.nn.relu(
        jnp.dot(z, fc1w_ref[...], preferred_element_type=jnp.float32)
        + fc1b_ref[...])
    logits = jnp.sum(t * fc2w_ref[...], axis=1, keepdims=True) + fc2b_ref[...]
    out_ref[...] = jax.nn.sigmoid(logits)


def _tc_head(ss, st, dsv, dtv, b2_row, fc1_w, fc1_b_row, fc2_w_row, fc2_b):
    return pl.pallas_call(
        _head_body,
        grid=(B // MB,),
        in_specs=[
            pl.BlockSpec((NF, MB, F), lambda m: (0, m, 0)),
            pl.BlockSpec((NF, MB, F), lambda m: (0, m, 0)),
            pl.BlockSpec((MB, 1), lambda m: (m, 0)),
            pl.BlockSpec((MB, 1), lambda m: (m, 0)),
            pl.BlockSpec((1, D), lambda m: (0, 0)),
            pl.BlockSpec((D, H1), lambda m: (0, 0)),
            pl.BlockSpec((1, H1), lambda m: (0, 0)),
            pl.BlockSpec((1, H1), lambda m: (0, 0)),
            pl.BlockSpec((1, 1), lambda m: (0, 0)),
        ],
        out_specs=pl.BlockSpec((MB, 1), lambda m: (m, 0)),
        out_shape=jax.ShapeDtypeStruct((B, 1), jnp.float32),
    )(ss, st, dsv, dtv, b2_row, fc1_w, fc1_b_row, fc2_w_row, fc2_b)


def kernel(x, edge_index, source_node, target_node,
           W1, b1, W2, b2, fc1_w, fc1_b, fc2_w, fc2_b):
    x_pad = jnp.pad(x, ((0, N_PAD - N), (0, 0)))
    npad = E_PAD - E
    fill = (N + (jnp.arange(npad, dtype=jnp.int32) % (N_PAD - N)))
    src2d = jnp.concatenate([edge_index[0], fill]).reshape(NEB, EB)
    dst2d = jnp.concatenate([edge_index[1], fill]).reshape(NEB, EB)
    sn2d = source_node.reshape(B // EB, EB)
    tn2d = target_node.reshape(B // EB, EB)

    deg = _sc_degree(dst2d)
    xs1, dinv_col = _tc_matmul_scale(x_pad, W1, deg.reshape(N_PAD, 1))
    dinv = dinv_col.reshape(N_PAD)
    s1 = _sc_aggregate(xs1, src2d, dst2d)
    xs2 = _tc_layer2(s1, dinv_col, b1.reshape(1, D), W2)
    s2 = _sc_aggregate(xs2, src2d, dst2d)
    ss, st, dsv, dtv = _sc_linkgather(s2, dinv, sn2d, tn2d)
    out = _tc_head(ss, st, dsv.reshape(B, 1), dtv.reshape(B, 1),
                   b2.reshape(1, D), fc1_w, fc1_b.reshape(1, H1),
                   fc2_w.reshape(1, H1), fc2_b.reshape(1, 1))
    return out.reshape(B)


# speedup vs baseline: 10.6561x; 1.0622x over previous
import functools

import jax
import jax.numpy as jnp
from jax import lax
from jax.experimental import pallas as pl
from jax.experimental.pallas import tpu as pltpu
from jax.experimental.pallas import tpu_sc as plsc

N, E, B, D, H1 = 10000, 100000, 8192, 768, 128
NC, NS, L = 2, 16, 16
N_PAD = 10240
NF = 6
F = D // NF
EB = 128
NBT = 52
E_PAD = NS * NBT * EB
RPT = N_PAD // NS
MB = 512

_MESH = plsc.VectorSubcoreMesh(
    core_axis_name="c", subcore_axis_name="s", num_cores=NC, num_subcores=NS)


def _deg_body(dst3d, deg_hbm, dst_v, ones_v, deg_v, hist_sh):
    cid = lax.axis_index("c")
    sid = lax.axis_index("s")
    for i in range(RPT // L):
        ones_v[pl.ds(L * i, L)] = jnp.full((L,), 1.0, jnp.float32)
    pltpu.sync_copy(ones_v, hist_sh.at[pl.ds(sid * RPT, RPT)])
    pltpu.sync_copy(dst3d.at[sid], dst_v)
    plsc.subcore_barrier()

    @pl.loop(0, NBT)
    def _(b):
        pltpu.sync_copy(ones_v.at[pl.ds(0, EB)],
                        hist_sh.at[dst_v.at[b]], add=True)

    plsc.subcore_barrier()
    @pl.when(cid == 0)
    def _():
        pltpu.sync_copy(hist_sh.at[pl.ds(sid * RPT, RPT)], deg_v)
        pltpu.sync_copy(deg_v, deg_hbm.at[pl.ds(sid * RPT, RPT)])


def _sc_degree(dst3d):
    return pl.kernel(
        _deg_body,
        out_type=jax.ShapeDtypeStruct((N_PAD,), jnp.float32),
        mesh=_MESH,
        scratch_types=[
            pltpu.VMEM((NBT, EB), jnp.int32),
            pltpu.VMEM((RPT,), jnp.float32),
            pltpu.VMEM((RPT,), jnp.float32),
            pltpu.VMEM_SHARED((N_PAD,), jnp.float32),
        ],
    )(dst3d)


NGRP = NBT // 2


def _agg_body(xs_hbm, src3d, dst3d, out_hbm,
              src_v, dst_v, bufs, gsems, ssems, chunk):
    cid = lax.axis_index("c")
    sid = lax.axis_index("s")
    pltpu.sync_copy(src3d.at[sid], src_v)
    pltpu.sync_copy(dst3d.at[sid], dst_v)
    nsub = RPT // EB
    for kk in range(NF // NC):
        k = cid * (NF // NC) + kk
        for i in range(nsub):
            r0 = sid * RPT + i * EB
            pltpu.sync_copy(xs_hbm.at[k, pl.ds(r0, EB)], bufs[0])
            pltpu.sync_copy(bufs[0], chunk.at[pl.ds(r0, EB)])
        plsc.subcore_barrier()

        def _gath(i, j):
            return pltpu.async_copy(xs_hbm.at[k].at[src_v.at[i]],
                                    bufs[j], gsems[j])

        def _gath_wait(i, j):
            pltpu.make_async_copy(xs_hbm.at[k].at[src_v.at[i]],
                                  bufs[j], gsems[j]).wait()

        def _scat(i, j):
            return pltpu.async_copy(bufs[j], chunk.at[dst_v.at[i]],
                                    ssems[j], add=True)

        def _scat_wait(i, j):
            pltpu.make_async_copy(bufs[j], chunk.at[dst_v.at[i]],
                                  ssems[j]).wait()

        _gath(0, 0)

        @pl.loop(0, NGRP)
        def _(g):
            for j in range(2):
                i = 2 * g + j
                jn = (j + 1) % 2
                if j == 0:
                    @pl.when(g > 0)
                    def _():
                        _scat_wait(i - 1, jn)
                    _gath(i + 1, jn)
                else:
                    _scat_wait(i - 1, jn)

                    @pl.when(g < NGRP - 1)
                    def _():
                        _gath(i + 1, jn)
                _gath_wait(i, j)
                _scat(i, j)

        _scat_wait(NBT - 1, 1)
        plsc.subcore_barrier()
        for i in range(nsub):
            r0 = sid * RPT + i * EB
            pltpu.sync_copy(chunk.at[pl.ds(r0, EB)], bufs[0])
            pltpu.sync_copy(bufs[0], out_hbm.at[k, pl.ds(r0, EB)])
        plsc.subcore_barrier()


def _sc_aggregate(xs, src3d, dst3d):
    return pl.kernel(
        _agg_body,
        out_type=jax.ShapeDtypeStruct((NF, N_PAD, F), jnp.float32),
        mesh=_MESH,
        scratch_types=[
            pltpu.VMEM((NBT, EB), jnp.int32),
            pltpu.VMEM((NBT, EB), jnp.int32),
            [pltpu.VMEM((EB, F), jnp.float32) for _ in range(2)],
            [pltpu.SemaphoreType.DMA for _ in range(2)],
            [pltpu.SemaphoreType.DMA for _ in range(2)],
            pltpu.VMEM_SHARED((N_PAD, F), jnp.float32),
        ],
    )(xs, src3d, dst3d)


def _link_body(s2, dinv_hbm, sn3d, tn3d, ss, st, dsv, dtv,
               idx_v, bufs, sems, dval_v):
    cid = lax.axis_index("c")
    sid = lax.axis_index("s")
    w = cid * NS + sid
    bpw = B // (NC * NS)
    nb = bpw // EB
    units = []
    for a in range(2):
        for b in range(nb):
            for k in range(NF):
                units.append((a, b, k))

    def _src(u):
        a, b, k = units[u]
        return (s2.at[k].at[idx_v.at[a * nb + b]], a, b, k)

    pltpu.sync_copy(sn3d.at[w], idx_v.at[pl.ds(0, nb)])
    pltpu.sync_copy(tn3d.at[w], idx_v.at[pl.ds(nb, nb)])
    for b in range(nb):
        r0 = w * bpw + b * EB
        pltpu.sync_copy(dinv_hbm.at[idx_v.at[b]], dval_v)
        pltpu.sync_copy(dval_v, dsv.at[pl.ds(r0, EB)])
        pltpu.sync_copy(dinv_hbm.at[idx_v.at[nb + b]], dval_v)
        pltpu.sync_copy(dval_v, dtv.at[pl.ds(r0, EB)])

    def _gath(u):
        ref, _, _, _ = _src(u)
        pltpu.async_copy(ref, bufs[u % 2], sems[u % 2])

    _gath(0)
    for u in range(len(units)):
        if u + 1 < len(units):
            _gath(u + 1)
        ref, a, b, k = _src(u)
        pltpu.make_async_copy(ref, bufs[u % 2], sems[u % 2]).wait()
        r0 = w * bpw + b * EB
        rows_out = (ss, st)[a]
        pltpu.sync_copy(bufs[u % 2], rows_out.at[k, pl.ds(r0, EB)])


def _sc_linkgather(s2, dinv, sn3d, tn3d):
    return pl.kernel(
        _link_body,
        out_type=(
            jax.ShapeDtypeStruct((NF, B, F), jnp.float32),
            jax.ShapeDtypeStruct((NF, B, F), jnp.float32),
            jax.ShapeDtypeStruct((B,), jnp.float32),
            jax.ShapeDtypeStruct((B,), jnp.float32),
        ),
        mesh=_MESH,
        scratch_types=[
            pltpu.VMEM((2 * (B // (NC * NS) // EB), EB), jnp.int32),
            [pltpu.VMEM((EB, F), jnp.float32) for _ in range(2)],
            [pltpu.SemaphoreType.DMA for _ in range(2)],
            pltpu.VMEM((EB,), jnp.float32),
        ],
    )(s2, dinv, sn3d, tn3d)


def _mm_scale_body(x_ref, w_ref, deg_ref, out_ref, dinv_ref):
    dinv = lax.rsqrt(deg_ref[...])
    dinv_ref[...] = dinv
    acc = jnp.dot(x_ref[...].astype(jnp.bfloat16),
                  w_ref[...].astype(jnp.bfloat16),
                  preferred_element_type=jnp.float32)
    acc = acc * dinv
    for k in range(NF):
        out_ref[k, :, :] = acc[:, k * F:(k + 1) * F]


def _tc_matmul_scale(x_pad, w, deg_col):
    return pl.pallas_call(
        _mm_scale_body,
        grid=(N_PAD // MB,),
        in_specs=[
            pl.BlockSpec((MB, D), lambda m: (m, 0)),
            pl.BlockSpec((D, D), lambda m: (0, 0)),
            pl.BlockSpec((MB, 1), lambda m: (m, 0)),
        ],
        out_specs=[
            pl.BlockSpec((NF, MB, F), lambda m: (0, m, 0)),
            pl.BlockSpec((MB, 1), lambda m: (m, 0)),
        ],
        out_shape=[
            jax.ShapeDtypeStruct((NF, N_PAD, F), jnp.float32),
            jax.ShapeDtypeStruct((N_PAD, 1), jnp.float32),
        ],
    )(x_pad, w, deg_col)


def _layer2_body(s_ref, dinv_ref, b1_ref, w_ref, out_ref):
    s = jnp.concatenate([s_ref[k] for k in range(NF)], axis=1)
    h = jax.nn.relu(dinv_ref[...] * s + b1_ref[...])
    acc = jnp.dot(h.astype(jnp.bfloat16),
                  w_ref[...].astype(jnp.bfloat16),
                  preferred_element_type=jnp.float32)
    acc = acc * dinv_ref[...]
    for k in range(NF):
        out_ref[k, :, :] = acc[:, k * F:(k + 1) * F]


def _tc_layer2(s1, dinv_col, b1_row, w2):
    return pl.pallas_call(
        _layer2_body,
        grid=(N_PAD // MB,),
        in_specs=[
            pl.BlockSpec((NF, MB, F), lambda m: (0, m, 0)),
            pl.BlockSpec((MB, 1), lambda m: (m, 0)),
            pl.BlockSpec((1, D), lambda m: (0, 0)),
            pl.BlockSpec((D, D), lambda m: (0, 0)),
        ],
        out_specs=pl.BlockSpec((NF, MB, F), lambda m: (0, m, 0)),
        out_shape=jax.ShapeDtypeStruct((NF, N_PAD, F), jnp.float32),
    )(s1, dinv_col, b1_row, w2)


def _head_body(ss_ref, st_ref, ds_ref, dt_ref, b2_ref, fc1w_ref, fc1b_ref,
               fc2w_ref, fc2b_ref, out_ref):
    sraw = jnp.concatenate([ss_ref[k] for k in range(NF)], axis=1)
    traw = jnp.concatenate([st_ref[k] for k in range(NF)], axis=1)
    hs = ds_ref[...] * sraw + b2_ref[...]
    ht = dt_ref[...] * traw + b2_ref[...]
    z = hs * ht
    t = jax.nn.relu(
        jnp.dot(z.astype(jnp.bfloat16), fc1w_ref[...].astype(jnp.bfloat16),
                preferred_element_type=jnp.float32)
        + fc1b_ref[...])
    logits = jnp.sum(t * fc2w_ref[...], axis=1, keepdims=True) + fc2b_ref[...]
    out_ref[...] = jax.nn.sigmoid(logits)


def _tc_head(ss, st, dsv, dtv, b2_row, fc1_w, fc1_b_row, fc2_w_row, fc2_b):
    return pl.pallas_call(
        _head_body,
        grid=(B // MB,),
        in_specs=[
            pl.BlockSpec((NF, MB, F), lambda m: (0, m, 0)),
            pl.BlockSpec((NF, MB, F), lambda m: (0, m, 0)),
            pl.BlockSpec((MB, 1), lambda m: (m, 0)),
            pl.BlockSpec((MB, 1), lambda m: (m, 0)),
            pl.BlockSpec((1, D), lambda m: (0, 0)),
            pl.BlockSpec((D, H1), lambda m: (0, 0)),
            pl.BlockSpec((1, H1), lambda m: (0, 0)),
            pl.BlockSpec((1, H1), lambda m: (0, 0)),
            pl.BlockSpec((1, 1), lambda m: (0, 0)),
        ],
        out_specs=pl.BlockSpec((MB, 1), lambda m: (m, 0)),
        out_shape=jax.ShapeDtypeStruct((B, 1), jnp.float32),
    )(ss, st, dsv, dtv, b2_row, fc1_w, fc1_b_row, fc2_w_row, fc2_b)


def kernel(x, edge_index, source_node, target_node,
           W1, b1, W2, b2, fc1_w, fc1_b, fc2_w, fc2_b):
    x_pad = jnp.pad(x, ((0, N_PAD - N), (0, 0)))
    ept = E // NS
    pad_t = NBT * EB - ept
    fill = (N + (jnp.arange(NS * pad_t, dtype=jnp.int32) % (N_PAD - N))
            ).reshape(NS, pad_t)

    def _edges3d(v):
        return jnp.concatenate(
            [v.reshape(NS, ept), fill], axis=1).reshape(NS, NBT, EB)

    src3d = _edges3d(edge_index[0])
    dst3d = _edges3d(edge_index[1])
    nbw = B // (NC * NS) // EB
    sn3d = source_node.reshape(NC * NS, nbw, EB)
    tn3d = target_node.reshape(NC * NS, nbw, EB)

    deg = _sc_degree(dst3d)
    xs1, dinv_col = _tc_matmul_scale(x_pad, W1, deg.reshape(N_PAD, 1))
    dinv = dinv_col.reshape(N_PAD)
    s1 = _sc_aggregate(xs1, src3d, dst3d)
    xs2 = _tc_layer2(s1, dinv_col, b1.reshape(1, D), W2)
    s2 = _sc_aggregate(xs2, src3d, dst3d)
    ss, st, dsv, dtv = _sc_linkgather(s2, dinv, sn3d, tn3d)
    out = _tc_head(ss, st, dsv.reshape(B, 1), dtv.reshape(B, 1),
                   b2.reshape(1, D), fc1_w, fc1_b.reshape(1, H1),
                   fc2_w.reshape(1, H1), fc2_b.reshape(1, 1))
    return out.reshape(B)


# speedup vs baseline: 11.4204x; 1.0717x over previous
import functools

import jax
import jax.numpy as jnp
from jax import lax
from jax.experimental import pallas as pl
from jax.experimental.pallas import tpu as pltpu
from jax.experimental.pallas import tpu_sc as plsc

N, E, B, D, H1 = 10000, 100000, 8192, 768, 128
NC, NS, L = 2, 16, 16
N_PAD = 10240
NF = 6
F = D // NF
EB = 128
NBT = 50
E_PAD = NS * NBT * EB
RPT = N_PAD // NS
MB = 512

_MESH = plsc.VectorSubcoreMesh(
    core_axis_name="c", subcore_axis_name="s", num_cores=NC, num_subcores=NS)


def _deg_body(dst3d, deg_hbm, dst_v, ones_v, deg_v, hist_sh):
    cid = lax.axis_index("c")
    sid = lax.axis_index("s")
    for i in range(RPT // L):
        ones_v[pl.ds(L * i, L)] = jnp.full((L,), 1.0, jnp.float32)
    pltpu.sync_copy(ones_v, hist_sh.at[pl.ds(sid * RPT, RPT)])
    pltpu.sync_copy(dst3d.at[sid], dst_v)
    plsc.subcore_barrier()

    @pl.loop(0, NBT)
    def _(b):
        pltpu.sync_copy(ones_v.at[pl.ds(0, EB)],
                        hist_sh.at[dst_v.at[b]], add=True)

    plsc.subcore_barrier()
    @pl.when(cid == 0)
    def _():
        pltpu.sync_copy(hist_sh.at[pl.ds(sid * RPT, RPT)], deg_v)
        pltpu.sync_copy(deg_v, deg_hbm.at[pl.ds(sid * RPT, RPT)])


def _sc_degree(dst3d):
    return pl.kernel(
        _deg_body,
        out_type=jax.ShapeDtypeStruct((N_PAD,), jnp.float32),
        mesh=_MESH,
        scratch_types=[
            pltpu.VMEM((NBT, EB), jnp.int32),
            pltpu.VMEM((RPT,), jnp.float32),
            pltpu.VMEM((RPT,), jnp.float32),
            pltpu.VMEM_SHARED((N_PAD,), jnp.float32),
        ],
    )(dst3d)


NGRP = NBT // 2


def _agg_body(xs_hbm, src3d, dst3d, out_hbm,
              src_v, dst_v, bufs, gsems, ssems, chunk):
    cid = lax.axis_index("c")
    sid = lax.axis_index("s")
    pltpu.sync_copy(src3d.at[sid], src_v)
    pltpu.sync_copy(dst3d.at[sid], dst_v)
    nsub = RPT // EB
    for kk in range(NF // NC):
        k = cid * (NF // NC) + kk
        for i in range(nsub):
            r0 = sid * RPT + i * EB
            pltpu.sync_copy(xs_hbm.at[k, pl.ds(r0, EB)], bufs[0])
            pltpu.sync_copy(bufs[0], chunk.at[pl.ds(r0, EB)])
        plsc.subcore_barrier()

        def _gath(i, j):
            return pltpu.async_copy(xs_hbm.at[k].at[src_v.at[i]],
                                    bufs[j], gsems[j])

        def _gath_wait(i, j):
            pltpu.make_async_copy(xs_hbm.at[k].at[src_v.at[i]],
                                  bufs[j], gsems[j]).wait()

        def _scat(i, j):
            return pltpu.async_copy(bufs[j], chunk.at[dst_v.at[i]],
                                    ssems[j], add=True)

        def _scat_wait(i, j):
            pltpu.make_async_copy(bufs[j], chunk.at[dst_v.at[i]],
                                  ssems[j]).wait()

        _gath(0, 0)

        @pl.loop(0, NGRP)
        def _(g):
            for j in range(2):
                i = 2 * g + j
                jn = (j + 1) % 2
                if j == 0:
                    @pl.when(g > 0)
                    def _():
                        _scat_wait(i - 1, jn)
                    _gath(i + 1, jn)
                else:
                    _scat_wait(i - 1, jn)

                    @pl.when(g < NGRP - 1)
                    def _():
                        _gath(i + 1, jn)
                _gath_wait(i, j)
                _scat(i, j)

        _scat_wait(NBT - 1, 1)
        plsc.subcore_barrier()
        for i in range(nsub):
            r0 = sid * RPT + i * EB
            pltpu.sync_copy(chunk.at[pl.ds(r0, EB)], bufs[0])
            pltpu.sync_copy(bufs[0], out_hbm.at[k, pl.ds(r0, EB)])
        plsc.subcore_barrier()


def _sc_aggregate(xs, src3d, dst3d):
    return pl.kernel(
        _agg_body,
        out_type=jax.ShapeDtypeStruct((NF, N_PAD, F), jnp.float32),
        mesh=_MESH,
        scratch_types=[
            pltpu.VMEM((NBT, EB), jnp.int32),
            pltpu.VMEM((NBT, EB), jnp.int32),
            [pltpu.VMEM((EB, F), jnp.float32) for _ in range(2)],
            [pltpu.SemaphoreType.DMA for _ in range(2)],
            [pltpu.SemaphoreType.DMA for _ in range(2)],
            pltpu.VMEM_SHARED((N_PAD, F), jnp.float32),
        ],
    )(xs, src3d, dst3d)


NBP = B // NS // EB


def _agg_link_body(xs_hbm, src3d, dst3d, dinv_hbm, sn3d, tn3d,
                   ss, st, dsv, dtv,
                   src_v, dst_v, bufs, gsems, ssems, idx_v, dval_v, chunk):
    cid = lax.axis_index("c")
    sid = lax.axis_index("s")
    pltpu.sync_copy(src3d.at[sid], src_v)
    pltpu.sync_copy(dst3d.at[sid], dst_v)
    pltpu.sync_copy(sn3d.at[sid], idx_v.at[pl.ds(0, NBP)])
    pltpu.sync_copy(tn3d.at[sid], idx_v.at[pl.ds(NBP, NBP)])

    @pl.when(cid == 0)
    def _():
        for b in range(NBP):
            pltpu.sync_copy(dinv_hbm.at[idx_v.at[b]], dval_v)
            pltpu.sync_copy(dval_v, dsv.at[pl.ds(sid * NBP * EB + b * EB,
                                                 EB)])

    @pl.when(cid == 1)
    def _():
        for b in range(NBP):
            pltpu.sync_copy(dinv_hbm.at[idx_v.at[NBP + b]], dval_v)
            pltpu.sync_copy(dval_v, dtv.at[pl.ds(sid * NBP * EB + b * EB,
                                                 EB)])

    nsub = RPT // EB
    for kk in range(NF // NC):
        k = cid * (NF // NC) + kk
        for i in range(nsub):
            r0 = sid * RPT + i * EB
            pltpu.sync_copy(xs_hbm.at[k, pl.ds(r0, EB)], bufs[0])
            pltpu.sync_copy(bufs[0], chunk.at[pl.ds(r0, EB)])
        plsc.subcore_barrier()

        def _gath(i, j):
            return pltpu.async_copy(xs_hbm.at[k].at[src_v.at[i]],
                                    bufs[j], gsems[j])

        def _gath_wait(i, j):
            pltpu.make_async_copy(xs_hbm.at[k].at[src_v.at[i]],
                                  bufs[j], gsems[j]).wait()

        def _scat(i, j):
            return pltpu.async_copy(bufs[j], chunk.at[dst_v.at[i]],
                                    ssems[j], add=True)

        def _scat_wait(i, j):
            pltpu.make_async_copy(bufs[j], chunk.at[dst_v.at[i]],
                                  ssems[j]).wait()

        _gath(0, 0)

        @pl.loop(0, NGRP)
        def _(g):
            for j in range(2):
                i = 2 * g + j
                jn = (j + 1) % 2
                if j == 0:
                    @pl.when(g > 0)
                    def _():
                        _scat_wait(i - 1, jn)
                    _gath(i + 1, jn)
                else:
                    _scat_wait(i - 1, jn)

                    @pl.when(g < NGRP - 1)
                    def _():
                        _gath(i + 1, jn)
                _gath_wait(i, j)
                _scat(i, j)

        _scat_wait(NBT - 1, 1)
        plsc.subcore_barrier()
        ub = 0
        for a, rows_out in ((0, ss), (1, st)):
            for b in range(NBP):
                jj = ub % 2
                pltpu.sync_copy(chunk.at[idx_v.at[a * NBP + b]], bufs[jj])
                pltpu.sync_copy(
                    bufs[jj],
                    rows_out.at[k, pl.ds(sid * NBP * EB + b * EB, EB)])
                ub += 1
        plsc.subcore_barrier()


def _sc_agg_link(xs, src3d, dst3d, dinv, sn3d, tn3d):
    return pl.kernel(
        _agg_link_body,
        out_type=(
            jax.ShapeDtypeStruct((NF, B, F), jnp.float32),
            jax.ShapeDtypeStruct((NF, B, F), jnp.float32),
            jax.ShapeDtypeStruct((B,), jnp.float32),
            jax.ShapeDtypeStruct((B,), jnp.float32),
        ),
        mesh=_MESH,
        scratch_types=[
            pltpu.VMEM((NBT, EB), jnp.int32),
            pltpu.VMEM((NBT, EB), jnp.int32),
            [pltpu.VMEM((EB, F), jnp.float32) for _ in range(2)],
            [pltpu.SemaphoreType.DMA for _ in range(2)],
            [pltpu.SemaphoreType.DMA for _ in range(2)],
            pltpu.VMEM((2 * NBP, EB), jnp.int32),
            pltpu.VMEM((EB,), jnp.float32),
            pltpu.VMEM_SHARED((N_PAD, F), jnp.float32),
        ],
    )(xs, src3d, dst3d, dinv, sn3d, tn3d)


def _mm_scale_body(x_ref, w_ref, deg_ref, out_ref, dinv_ref):
    dinv = lax.rsqrt(deg_ref[...])
    dinv_ref[...] = dinv
    acc = jnp.dot(x_ref[...].astype(jnp.bfloat16),
                  w_ref[...].astype(jnp.bfloat16),
                  preferred_element_type=jnp.float32)
    acc = acc * dinv
    for k in range(NF):
        out_ref[k, :, :] = acc[:, k * F:(k + 1) * F]


def _tc_matmul_scale(x_pad, w, deg_col):
    return pl.pallas_call(
        _mm_scale_body,
        grid=(N_PAD // MB,),
        in_specs=[
            pl.BlockSpec((MB, D), lambda m: (m, 0)),
            pl.BlockSpec((D, D), lambda m: (0, 0)),
            pl.BlockSpec((MB, 1), lambda m: (m, 0)),
        ],
        out_specs=[
            pl.BlockSpec((NF, MB, F), lambda m: (0, m, 0)),
            pl.BlockSpec((MB, 1), lambda m: (m, 0)),
        ],
        out_shape=[
            jax.ShapeDtypeStruct((NF, N_PAD, F), jnp.float32),
            jax.ShapeDtypeStruct((N_PAD, 1), jnp.float32),
        ],
    )(x_pad, w, deg_col)


def _layer2_body(s_ref, dinv_ref, b1_ref, w_ref, out_ref):
    s = jnp.concatenate([s_ref[k] for k in range(NF)], axis=1)
    h = jax.nn.relu(dinv_ref[...] * s + b1_ref[...])
    acc = jnp.dot(h.astype(jnp.bfloat16),
                  w_ref[...].astype(jnp.bfloat16),
                  preferred_element_type=jnp.float32)
    acc = acc * dinv_ref[...]
    for k in range(NF):
        out_ref[k, :, :] = acc[:, k * F:(k + 1) * F]


def _tc_layer2(s1, dinv_col, b1_row, w2):
    return pl.pallas_call(
        _layer2_body,
        grid=(N_PAD // MB,),
        in_specs=[
            pl.BlockSpec((NF, MB, F), lambda m: (0, m, 0)),
            pl.BlockSpec((MB, 1), lambda m: (m, 0)),
            pl.BlockSpec((1, D), lambda m: (0, 0)),
            pl.BlockSpec((D, D), lambda m: (0, 0)),
        ],
        out_specs=pl.BlockSpec((NF, MB, F), lambda m: (0, m, 0)),
        out_shape=jax.ShapeDtypeStruct((NF, N_PAD, F), jnp.float32),
    )(s1, dinv_col, b1_row, w2)


def _head_body(ss_ref, st_ref, ds_ref, dt_ref, b2_ref, fc1w_ref, fc1b_ref,
               fc2w_ref, fc2b_ref, out_ref):
    sraw = jnp.concatenate([ss_ref[k] for k in range(NF)], axis=1)
    traw = jnp.concatenate([st_ref[k] for k in range(NF)], axis=1)
    hs = ds_ref[...] * sraw + b2_ref[...]
    ht = dt_ref[...] * traw + b2_ref[...]
    z = hs * ht
    t = jax.nn.relu(
        jnp.dot(z.astype(jnp.bfloat16), fc1w_ref[...].astype(jnp.bfloat16),
                preferred_element_type=jnp.float32)
        + fc1b_ref[...])
    logits = jnp.sum(t * fc2w_ref[...], axis=1, keepdims=True) + fc2b_ref[...]
    out_ref[...] = jax.nn.sigmoid(logits)


def _tc_head(ss, st, dsv, dtv, b2_row, fc1_w, fc1_b_row, fc2_w_row, fc2_b):
    return pl.pallas_call(
        _head_body,
        grid=(B // MB,),
        in_specs=[
            pl.BlockSpec((NF, MB, F), lambda m: (0, m, 0)),
            pl.BlockSpec((NF, MB, F), lambda m: (0, m, 0)),
            pl.BlockSpec((MB, 1), lambda m: (m, 0)),
            pl.BlockSpec((MB, 1), lambda m: (m, 0)),
            pl.BlockSpec((1, D), lambda m: (0, 0)),
            pl.BlockSpec((D, H1), lambda m: (0, 0)),
            pl.BlockSpec((1, H1), lambda m: (0, 0)),
            pl.BlockSpec((1, H1), lambda m: (0, 0)),
            pl.BlockSpec((1, 1), lambda m: (0, 0)),
        ],
        out_specs=pl.BlockSpec((MB, 1), lambda m: (m, 0)),
        out_shape=jax.ShapeDtypeStruct((B, 1), jnp.float32),
    )(ss, st, dsv, dtv, b2_row, fc1_w, fc1_b_row, fc2_w_row, fc2_b)


def kernel(x, edge_index, source_node, target_node,
           W1, b1, W2, b2, fc1_w, fc1_b, fc2_w, fc2_b):
    x_pad = jnp.pad(x, ((0, N_PAD - N), (0, 0)))
    ept = E // NS
    pad_t = NBT * EB - ept
    fill = (N + (jnp.arange(NS * pad_t, dtype=jnp.int32) % (N_PAD - N))
            ).reshape(NS, pad_t)

    def _edges3d(v):
        return jnp.concatenate(
            [v.reshape(NS, ept), fill], axis=1).reshape(NS, NBT, EB)

    src3d = _edges3d(edge_index[0])
    dst3d = _edges3d(edge_index[1])
    sn3d = source_node.reshape(NS, NBP, EB)
    tn3d = target_node.reshape(NS, NBP, EB)

    deg = _sc_degree(dst3d)
    xs1, dinv_col = _tc_matmul_scale(x_pad, W1, deg.reshape(N_PAD, 1))
    dinv = dinv_col.reshape(N_PAD)
    s1 = _sc_aggregate(xs1, src3d, dst3d)
    xs2 = _tc_layer2(s1, dinv_col, b1.reshape(1, D), W2)
    ss, st, dsv, dtv = _sc_agg_link(xs2, src3d, dst3d, dinv, sn3d, tn3d)
    out = _tc_head(ss, st, dsv.reshape(B, 1), dtv.reshape(B, 1),
                   b2.reshape(1, D), fc1_w, fc1_b.reshape(1, H1),
                   fc2_w.reshape(1, H1), fc2_b.reshape(1, 1))
    return out.reshape(B)


# speedup vs baseline: 11.7263x; 1.0268x over previous
import functools

import jax
import jax.numpy as jnp
from jax import lax
from jax.experimental import pallas as pl
from jax.experimental.pallas import tpu as pltpu
from jax.experimental.pallas import tpu_sc as plsc

N, E, B, D, H1 = 10000, 100000, 8192, 768, 128
NC, NS, L = 2, 16, 16
N_PAD = 10240
NF = 6
F = D // NF
EB = 128
NBT = 50
E_PAD = NS * NBT * EB
RPT = N_PAD // NS
MB = 512

_MESH = plsc.VectorSubcoreMesh(
    core_axis_name="c", subcore_axis_name="s", num_cores=NC, num_subcores=NS)


def _deg_body(dst3d, deg_hbm, dst_v, ones_v, deg_v, hist_sh):
    cid = lax.axis_index("c")
    sid = lax.axis_index("s")
    for i in range(RPT // L):
        ones_v[pl.ds(L * i, L)] = jnp.full((L,), 1.0, jnp.float32)
    pltpu.sync_copy(ones_v, hist_sh.at[pl.ds(sid * RPT, RPT)])
    pltpu.sync_copy(dst3d.at[sid], dst_v)
    plsc.subcore_barrier()

    @pl.loop(0, NBT)
    def _(b):
        pltpu.sync_copy(ones_v.at[pl.ds(0, EB)],
                        hist_sh.at[dst_v.at[b]], add=True)

    plsc.subcore_barrier()
    @pl.when(cid == 0)
    def _():
        pltpu.sync_copy(hist_sh.at[pl.ds(sid * RPT, RPT)], deg_v)
        pltpu.sync_copy(deg_v, deg_hbm.at[pl.ds(sid * RPT, RPT)])


def _sc_degree(dst3d):
    return pl.kernel(
        _deg_body,
        out_type=jax.ShapeDtypeStruct((N_PAD,), jnp.float32),
        mesh=_MESH,
        scratch_types=[
            pltpu.VMEM((NBT, EB), jnp.int32),
            pltpu.VMEM((RPT,), jnp.float32),
            pltpu.VMEM((RPT,), jnp.float32),
            pltpu.VMEM_SHARED((N_PAD,), jnp.float32),
        ],
    )(dst3d)


NGRP = NBT // 2


def _agg_body(xs_hbm, src3d, dst3d, out_hbm,
              src_v, dst_v, bufs, gsems, ssems, chunk):
    cid = lax.axis_index("c")
    sid = lax.axis_index("s")
    pltpu.sync_copy(src3d.at[sid], src_v)
    pltpu.sync_copy(dst3d.at[sid], dst_v)
    for kk in range(NF // NC):
        k = cid * (NF // NC) + kk
        r0 = sid * RPT
        pltpu.sync_copy(xs_hbm.at[k, pl.ds(r0, RPT)],
                        chunk.at[pl.ds(r0, RPT)])
        plsc.subcore_barrier()

        def _gath(i, j):
            return pltpu.async_copy(xs_hbm.at[k].at[src_v.at[i]],
                                    bufs[j], gsems[j])

        def _gath_wait(i, j):
            pltpu.make_async_copy(xs_hbm.at[k].at[src_v.at[i]],
                                  bufs[j], gsems[j]).wait()

        def _scat(i, j):
            return pltpu.async_copy(bufs[j], chunk.at[dst_v.at[i]],
                                    ssems[j], add=True)

        def _scat_wait(i, j):
            pltpu.make_async_copy(bufs[j], chunk.at[dst_v.at[i]],
                                  ssems[j]).wait()

        _gath(0, 0)

        @pl.loop(0, NGRP)
        def _(g):
            for j in range(2):
                i = 2 * g + j
                jn = (j + 1) % 2
                if j == 0:
                    @pl.when(g > 0)
                    def _():
                        _scat_wait(i - 1, jn)
                    _gath(i + 1, jn)
                else:
                    _scat_wait(i - 1, jn)

                    @pl.when(g < NGRP - 1)
                    def _():
                        _gath(i + 1, jn)
                _gath_wait(i, j)
                _scat(i, j)

        _scat_wait(NBT - 1, 1)
        plsc.subcore_barrier()
        pltpu.sync_copy(chunk.at[pl.ds(r0, RPT)],
                        out_hbm.at[k, pl.ds(r0, RPT)])
        plsc.subcore_barrier()


def _sc_aggregate(xs, src3d, dst3d):
    return pl.kernel(
        _agg_body,
        out_type=jax.ShapeDtypeStruct((NF, N_PAD, F), jnp.float32),
        mesh=_MESH,
        scratch_types=[
            pltpu.VMEM((NBT, EB), jnp.int32),
            pltpu.VMEM((NBT, EB), jnp.int32),
            [pltpu.VMEM((EB, F), jnp.float32) for _ in range(2)],
            [pltpu.SemaphoreType.DMA for _ in range(2)],
            [pltpu.SemaphoreType.DMA for _ in range(2)],
            pltpu.VMEM_SHARED((N_PAD, F), jnp.float32),
        ],
    )(xs, src3d, dst3d)


NBP = B // NS // EB


def _agg_link_body(xs_hbm, src3d, dst3d, dinv_hbm, sn3d, tn3d,
                   ss, st, dsv, dtv,
                   src_v, dst_v, bufs, gsems, ssems, idx_v, dval_v, chunk):
    cid = lax.axis_index("c")
    sid = lax.axis_index("s")
    pltpu.sync_copy(src3d.at[sid], src_v)
    pltpu.sync_copy(dst3d.at[sid], dst_v)
    pltpu.sync_copy(sn3d.at[sid], idx_v.at[pl.ds(0, NBP)])
    pltpu.sync_copy(tn3d.at[sid], idx_v.at[pl.ds(NBP, NBP)])

    @pl.when(cid == 0)
    def _():
        for b in range(NBP):
            pltpu.sync_copy(dinv_hbm.at[idx_v.at[b]], dval_v)
            pltpu.sync_copy(dval_v, dsv.at[pl.ds(sid * NBP * EB + b * EB,
                                                 EB)])

    @pl.when(cid == 1)
    def _():
        for b in range(NBP):
            pltpu.sync_copy(dinv_hbm.at[idx_v.at[NBP + b]], dval_v)
            pltpu.sync_copy(dval_v, dtv.at[pl.ds(sid * NBP * EB + b * EB,
                                                 EB)])

    for kk in range(NF // NC):
        k = cid * (NF // NC) + kk
        r0 = sid * RPT
        pltpu.sync_copy(xs_hbm.at[k, pl.ds(r0, RPT)],
                        chunk.at[pl.ds(r0, RPT)])
        plsc.subcore_barrier()

        def _gath(i, j):
            return pltpu.async_copy(xs_hbm.at[k].at[src_v.at[i]],
                                    bufs[j], gsems[j])

        def _gath_wait(i, j):
            pltpu.make_async_copy(xs_hbm.at[k].at[src_v.at[i]],
                                  bufs[j], gsems[j]).wait()

        def _scat(i, j):
            return pltpu.async_copy(bufs[j], chunk.at[dst_v.at[i]],
                                    ssems[j], add=True)

        def _scat_wait(i, j):
            pltpu.make_async_copy(bufs[j], chunk.at[dst_v.at[i]],
                                  ssems[j]).wait()

        _gath(0, 0)

        @pl.loop(0, NGRP)
        def _(g):
            for j in range(2):
                i = 2 * g + j
                jn = (j + 1) % 2
                if j == 0:
                    @pl.when(g > 0)
                    def _():
                        _scat_wait(i - 1, jn)
                    _gath(i + 1, jn)
                else:
                    _scat_wait(i - 1, jn)

                    @pl.when(g < NGRP - 1)
                    def _():
                        _gath(i + 1, jn)
                _gath_wait(i, j)
                _scat(i, j)

        _scat_wait(NBT - 1, 1)
        plsc.subcore_barrier()
        ub = 0
        for a, rows_out in ((0, ss), (1, st)):
            for b in range(NBP):
                jj = ub % 2
                pltpu.sync_copy(chunk.at[idx_v.at[a * NBP + b]], bufs[jj])
                pltpu.sync_copy(
                    bufs[jj],
                    rows_out.at[k, pl.ds(sid * NBP * EB + b * EB, EB)])
                ub += 1
        plsc.subcore_barrier()


def _sc_agg_link(xs, src3d, dst3d, dinv, sn3d, tn3d):
    return pl.kernel(
        _agg_link_body,
        out_type=(
            jax.ShapeDtypeStruct((NF, B, F), jnp.float32),
            jax.ShapeDtypeStruct((NF, B, F), jnp.float32),
            jax.ShapeDtypeStruct((B,), jnp.float32),
            jax.ShapeDtypeStruct((B,), jnp.float32),
        ),
        mesh=_MESH,
        scratch_types=[
            pltpu.VMEM((NBT, EB), jnp.int32),
            pltpu.VMEM((NBT, EB), jnp.int32),
            [pltpu.VMEM((EB, F), jnp.float32) for _ in range(2)],
            [pltpu.SemaphoreType.DMA for _ in range(2)],
            [pltpu.SemaphoreType.DMA for _ in range(2)],
            pltpu.VMEM((2 * NBP, EB), jnp.int32),
            pltpu.VMEM((EB,), jnp.float32),
            pltpu.VMEM_SHARED((N_PAD, F), jnp.float32),
        ],
    )(xs, src3d, dst3d, dinv, sn3d, tn3d)


def _mm_scale_body(x_ref, w_ref, deg_ref, out_ref, dinv_ref):
    dinv = lax.rsqrt(deg_ref[...])
    dinv_ref[...] = dinv
    acc = jnp.dot(x_ref[...].astype(jnp.bfloat16),
                  w_ref[...].astype(jnp.bfloat16),
                  preferred_element_type=jnp.float32)
    acc = acc * dinv
    for k in range(NF):
        out_ref[k, :, :] = acc[:, k * F:(k + 1) * F]


def _tc_matmul_scale(x_pad, w, deg_col):
    return pl.pallas_call(
        _mm_scale_body,
        grid=(N_PAD // MB,),
        in_specs=[
            pl.BlockSpec((MB, D), lambda m: (m, 0)),
            pl.BlockSpec((D, D), lambda m: (0, 0)),
            pl.BlockSpec((MB, 1), lambda m: (m, 0)),
        ],
        out_specs=[
            pl.BlockSpec((NF, MB, F), lambda m: (0, m, 0)),
            pl.BlockSpec((MB, 1), lambda m: (m, 0)),
        ],
        out_shape=[
            jax.ShapeDtypeStruct((NF, N_PAD, F), jnp.float32),
            jax.ShapeDtypeStruct((N_PAD, 1), jnp.float32),
        ],
    )(x_pad, w, deg_col)


def _layer2_body(s_ref, dinv_ref, b1_ref, w_ref, out_ref):
    s = jnp.concatenate([s_ref[k] for k in range(NF)], axis=1)
    h = jax.nn.relu(dinv_ref[...] * s + b1_ref[...])
    acc = jnp.dot(h.astype(jnp.bfloat16),
                  w_ref[...].astype(jnp.bfloat16),
                  preferred_element_type=jnp.float32)
    acc = acc * dinv_ref[...]
    for k in range(NF):
        out_ref[k, :, :] = acc[:, k * F:(k + 1) * F]


def _tc_layer2(s1, dinv_col, b1_row, w2):
    return pl.pallas_call(
        _layer2_body,
        grid=(N_PAD // MB,),
        in_specs=[
            pl.BlockSpec((NF, MB, F), lambda m: (0, m, 0)),
            pl.BlockSpec((MB, 1), lambda m: (m, 0)),
            pl.BlockSpec((1, D), lambda m: (0, 0)),
            pl.BlockSpec((D, D), lambda m: (0, 0)),
        ],
        out_specs=pl.BlockSpec((NF, MB, F), lambda m: (0, m, 0)),
        out_shape=jax.ShapeDtypeStruct((NF, N_PAD, F), jnp.float32),
    )(s1, dinv_col, b1_row, w2)


def _head_body(ss_ref, st_ref, ds_ref, dt_ref, b2_ref, fc1w_ref, fc1b_ref,
               fc2w_ref, fc2b_ref, out_ref):
    sraw = jnp.concatenate([ss_ref[k] for k in range(NF)], axis=1)
    traw = jnp.concatenate([st_ref[k] for k in range(NF)], axis=1)
    hs = ds_ref[...] * sraw + b2_ref[...]
    ht = dt_ref[...] * traw + b2_ref[...]
    z = hs * ht
    t = jax.nn.relu(
        jnp.dot(z.astype(jnp.bfloat16), fc1w_ref[...].astype(jnp.bfloat16),
                preferred_element_type=jnp.float32)
        + fc1b_ref[...])
    logits = jnp.sum(t * fc2w_ref[...], axis=1, keepdims=True) + fc2b_ref[...]
    out_ref[...] = jax.nn.sigmoid(logits)


def _tc_head(ss, st, dsv, dtv, b2_row, fc1_w, fc1_b_row, fc2_w_row, fc2_b):
    return pl.pallas_call(
        _head_body,
        grid=(B // MB,),
        in_specs=[
            pl.BlockSpec((NF, MB, F), lambda m: (0, m, 0)),
            pl.BlockSpec((NF, MB, F), lambda m: (0, m, 0)),
            pl.BlockSpec((MB, 1), lambda m: (m, 0)),
            pl.BlockSpec((MB, 1), lambda m: (m, 0)),
            pl.BlockSpec((1, D), lambda m: (0, 0)),
            pl.BlockSpec((D, H1), lambda m: (0, 0)),
            pl.BlockSpec((1, H1), lambda m: (0, 0)),
            pl.BlockSpec((1, H1), lambda m: (0, 0)),
            pl.BlockSpec((1, 1), lambda m: (0, 0)),
        ],
        out_specs=pl.BlockSpec((MB, 1), lambda m: (m, 0)),
        out_shape=jax.ShapeDtypeStruct((B, 1), jnp.float32),
    )(ss, st, dsv, dtv, b2_row, fc1_w, fc1_b_row, fc2_w_row, fc2_b)


def kernel(x, edge_index, source_node, target_node,
           W1, b1, W2, b2, fc1_w, fc1_b, fc2_w, fc2_b):
    x_pad = jnp.pad(x, ((0, N_PAD - N), (0, 0)))
    ept = E // NS
    pad_t = NBT * EB - ept
    fill = (N + (jnp.arange(NS * pad_t, dtype=jnp.int32) % (N_PAD - N))
            ).reshape(NS, pad_t)

    def _edges3d(v):
        return jnp.concatenate(
            [v.reshape(NS, ept), fill], axis=1).reshape(NS, NBT, EB)

    src3d = _edges3d(edge_index[0])
    dst3d = _edges3d(edge_index[1])
    sn3d = source_node.reshape(NS, NBP, EB)
    tn3d = target_node.reshape(NS, NBP, EB)

    deg = _sc_degree(dst3d)
    xs1, dinv_col = _tc_matmul_scale(x_pad, W1, deg.reshape(N_PAD, 1))
    dinv = dinv_col.reshape(N_PAD)
    s1 = _sc_aggregate(xs1, src3d, dst3d)
    xs2 = _tc_layer2(s1, dinv_col, b1.reshape(1, D), W2)
    ss, st, dsv, dtv = _sc_agg_link(xs2, src3d, dst3d, dinv, sn3d, tn3d)
    out = _tc_head(ss, st, dsv.reshape(B, 1), dtv.reshape(B, 1),
                   b2.reshape(1, D), fc1_w, fc1_b.reshape(1, H1),
                   fc2_w.reshape(1, H1), fc2_b.reshape(1, 1))
    return out.reshape(B)


# speedup vs baseline: 11.7465x; 1.0017x over previous
import jax
import jax.numpy as jnp
from jax import lax
from jax.experimental import pallas as pl
from jax.experimental.pallas import tpu as pltpu
from jax.experimental.pallas import tpu_sc as plsc

N, E, B, D, H1 = 10000, 100000, 8192, 768, 128
NC, NS, L = 2, 16, 16
N_PAD = 10240
NF = 6
F = D // NF
EB = 128
NBT = 50
E_PAD = NS * NBT * EB
RPT = N_PAD // NS
MB = 512

_MESH = plsc.VectorSubcoreMesh(
    core_axis_name="c", subcore_axis_name="s", num_cores=NC, num_subcores=NS)


def _deg_body(dst3d, deg_hbm, dst_v, ones_v, deg_v, hist_sh):
    cid = lax.axis_index("c")
    sid = lax.axis_index("s")
    for i in range(RPT // L):
        ones_v[pl.ds(L * i, L)] = jnp.full((L,), 1.0, jnp.float32)
    pltpu.sync_copy(ones_v, hist_sh.at[pl.ds(sid * RPT, RPT)])
    pltpu.sync_copy(dst3d.at[sid], dst_v)
    plsc.subcore_barrier()

    @pl.loop(0, NBT)
    def _(b):
        pltpu.sync_copy(ones_v.at[pl.ds(0, EB)],
                        hist_sh.at[dst_v.at[b]], add=True)

    plsc.subcore_barrier()
    @pl.when(cid == 0)
    def _():
        pltpu.sync_copy(hist_sh.at[pl.ds(sid * RPT, RPT)], deg_v)
        pltpu.sync_copy(deg_v, deg_hbm.at[pl.ds(sid * RPT, RPT)])


def _sc_degree(dst3d):
    return pl.kernel(
        _deg_body,
        out_type=jax.ShapeDtypeStruct((N_PAD,), jnp.float32),
        mesh=_MESH,
        scratch_types=[
            pltpu.VMEM((NBT, EB), jnp.int32),
            pltpu.VMEM((RPT,), jnp.float32),
            pltpu.VMEM((RPT,), jnp.float32),
            pltpu.VMEM_SHARED((N_PAD,), jnp.float32),
        ],
    )(dst3d)


NGRP = NBT // 2


def _agg_body(xs_hbm, src3d, dst3d, out_hbm,
              src_v, dst_v, bufs, gsems, ssems, chunk):
    cid = lax.axis_index("c")
    sid = lax.axis_index("s")
    pltpu.sync_copy(src3d.at[sid], src_v)
    pltpu.sync_copy(dst3d.at[sid], dst_v)
    for kk in range(NF // NC):
        k = cid * (NF // NC) + kk
        r0 = sid * RPT
        pltpu.sync_copy(xs_hbm.at[k, pl.ds(r0, RPT)],
                        chunk.at[pl.ds(r0, RPT)])
        plsc.subcore_barrier()

        def _gath(i, j):
            return pltpu.async_copy(xs_hbm.at[k].at[src_v.at[i]],
                                    bufs[j], gsems[j])

        def _gath_wait(i, j):
            pltpu.make_async_copy(xs_hbm.at[k].at[src_v.at[i]],
                                  bufs[j], gsems[j]).wait()

        def _scat(i, j):
            return pltpu.async_copy(bufs[j], chunk.at[dst_v.at[i]],
                                    ssems[j], add=True)

        def _scat_wait(i, j):
            pltpu.make_async_copy(bufs[j], chunk.at[dst_v.at[i]],
                                  ssems[j]).wait()

        _gath(0, 0)

        @pl.loop(0, NGRP)
        def _(g):
            for j in range(2):
                i = 2 * g + j
                jn = (j + 1) % 2
                if j == 0:
                    @pl.when(g > 0)
                    def _():
                        _scat_wait(i - 1, jn)
                    _gath(i + 1, jn)
                else:
                    _scat_wait(i - 1, jn)

                    @pl.when(g < NGRP - 1)
                    def _():
                        _gath(i + 1, jn)
                _gath_wait(i, j)
                _scat(i, j)

        _scat_wait(NBT - 1, 1)
        plsc.subcore_barrier()
        pltpu.sync_copy(chunk.at[pl.ds(r0, RPT)],
                        out_hbm.at[k, pl.ds(r0, RPT)])
        plsc.subcore_barrier()


def _sc_aggregate(xs, src3d, dst3d):
    return pl.kernel(
        _agg_body,
        out_type=jax.ShapeDtypeStruct((NF, N_PAD, F), jnp.float32),
        mesh=_MESH,
        scratch_types=[
            pltpu.VMEM((NBT, EB), jnp.int32),
            pltpu.VMEM((NBT, EB), jnp.int32),
            [pltpu.VMEM((EB, F), jnp.float32) for _ in range(2)],
            [pltpu.SemaphoreType.DMA for _ in range(2)],
            [pltpu.SemaphoreType.DMA for _ in range(2)],
            pltpu.VMEM_SHARED((N_PAD, F), jnp.float32),
        ],
    )(xs, src3d, dst3d)


NBP = B // NS // EB


def _agg_link_body(xs_hbm, src3d, dst3d, dinv_hbm, sn3d, tn3d,
                   ss, st, dsv, dtv,
                   src_v, dst_v, bufs, gsems, ssems, idx_v, dval_v, chunk):
    cid = lax.axis_index("c")
    sid = lax.axis_index("s")
    pltpu.sync_copy(src3d.at[sid], src_v)
    pltpu.sync_copy(dst3d.at[sid], dst_v)
    pltpu.sync_copy(sn3d.at[sid], idx_v.at[pl.ds(0, NBP)])
    pltpu.sync_copy(tn3d.at[sid], idx_v.at[pl.ds(NBP, NBP)])

    @pl.when(cid == 0)
    def _():
        for b in range(NBP):
            pltpu.sync_copy(dinv_hbm.at[idx_v.at[b]], dval_v)
            pltpu.sync_copy(dval_v, dsv.at[pl.ds(sid * NBP * EB + b * EB,
                                                 EB)])

    @pl.when(cid == 1)
    def _():
        for b in range(NBP):
            pltpu.sync_copy(dinv_hbm.at[idx_v.at[NBP + b]], dval_v)
            pltpu.sync_copy(dval_v, dtv.at[pl.ds(sid * NBP * EB + b * EB,
                                                 EB)])

    for kk in range(NF // NC):
        k = cid * (NF // NC) + kk
        r0 = sid * RPT
        pltpu.sync_copy(xs_hbm.at[k, pl.ds(r0, RPT)],
                        chunk.at[pl.ds(r0, RPT)])
        plsc.subcore_barrier()

        def _gath(i, j):
            return pltpu.async_copy(xs_hbm.at[k].at[src_v.at[i]],
                                    bufs[j], gsems[j])

        def _gath_wait(i, j):
            pltpu.make_async_copy(xs_hbm.at[k].at[src_v.at[i]],
                                  bufs[j], gsems[j]).wait()

        def _scat(i, j):
            return pltpu.async_copy(bufs[j], chunk.at[dst_v.at[i]],
                                    ssems[j], add=True)

        def _scat_wait(i, j):
            pltpu.make_async_copy(bufs[j], chunk.at[dst_v.at[i]],
                                  ssems[j]).wait()

        _gath(0, 0)

        @pl.loop(0, NGRP)
        def _(g):
            for j in range(2):
                i = 2 * g + j
                jn = (j + 1) % 2
                if j == 0:
                    @pl.when(g > 0)
                    def _():
                        _scat_wait(i - 1, jn)
                    _gath(i + 1, jn)
                else:
                    _scat_wait(i - 1, jn)

                    @pl.when(g < NGRP - 1)
                    def _():
                        _gath(i + 1, jn)
                _gath_wait(i, j)
                _scat(i, j)

        _scat_wait(NBT - 1, 1)
        plsc.subcore_barrier()
        ub = 0
        for a, rows_out in ((0, ss), (1, st)):
            for b in range(NBP):
                jj = ub % 2
                pltpu.sync_copy(chunk.at[idx_v.at[a * NBP + b]], bufs[jj])
                pltpu.sync_copy(
                    bufs[jj],
                    rows_out.at[k, pl.ds(sid * NBP * EB + b * EB, EB)])
                ub += 1
        plsc.subcore_barrier()


def _sc_agg_link(xs, src3d, dst3d, dinv, sn3d, tn3d):
    return pl.kernel(
        _agg_link_body,
        out_type=(
            jax.ShapeDtypeStruct((NF, B, F), jnp.float32),
            jax.ShapeDtypeStruct((NF, B, F), jnp.float32),
            jax.ShapeDtypeStruct((B,), jnp.float32),
            jax.ShapeDtypeStruct((B,), jnp.float32),
        ),
        mesh=_MESH,
        scratch_types=[
            pltpu.VMEM((NBT, EB), jnp.int32),
            pltpu.VMEM((NBT, EB), jnp.int32),
            [pltpu.VMEM((EB, F), jnp.float32) for _ in range(2)],
            [pltpu.SemaphoreType.DMA for _ in range(2)],
            [pltpu.SemaphoreType.DMA for _ in range(2)],
            pltpu.VMEM((2 * NBP, EB), jnp.int32),
            pltpu.VMEM((EB,), jnp.float32),
            pltpu.VMEM_SHARED((N_PAD, F), jnp.float32),
        ],
    )(xs, src3d, dst3d, dinv, sn3d, tn3d)


def _mm_scale_body(x_ref, w_ref, deg_ref, out_ref, dinv_ref):
    dinv = lax.rsqrt(deg_ref[...])
    dinv_ref[...] = dinv
    acc = jnp.dot(x_ref[...].astype(jnp.bfloat16),
                  w_ref[...].astype(jnp.bfloat16),
                  preferred_element_type=jnp.float32)
    acc = acc * dinv
    for k in range(NF):
        out_ref[k, :, :] = acc[:, k * F:(k + 1) * F]


def _tc_matmul_scale(x_pad, w, deg_col):
    return pl.pallas_call(
        _mm_scale_body,
        grid=(N_PAD // MB,),
        in_specs=[
            pl.BlockSpec((MB, D), lambda m: (m, 0)),
            pl.BlockSpec((D, D), lambda m: (0, 0)),
            pl.BlockSpec((MB, 1), lambda m: (m, 0)),
        ],
        out_specs=[
            pl.BlockSpec((NF, MB, F), lambda m: (0, m, 0)),
            pl.BlockSpec((MB, 1), lambda m: (m, 0)),
        ],
        out_shape=[
            jax.ShapeDtypeStruct((NF, N_PAD, F), jnp.float32),
            jax.ShapeDtypeStruct((N_PAD, 1), jnp.float32),
        ],
    )(x_pad, w, deg_col)


def _layer2_body(s_ref, dinv_ref, b1_ref, w_ref, out_ref):
    s = jnp.concatenate([s_ref[k] for k in range(NF)], axis=1)
    h = jax.nn.relu(dinv_ref[...] * s + b1_ref[...])
    acc = jnp.dot(h.astype(jnp.bfloat16),
                  w_ref[...].astype(jnp.bfloat16),
                  preferred_element_type=jnp.float32)
    acc = acc * dinv_ref[...]
    for k in range(NF):
        out_ref[k, :, :] = acc[:, k * F:(k + 1) * F]


def _tc_layer2(s1, dinv_col, b1_row, w2):
    return pl.pallas_call(
        _layer2_body,
        grid=(N_PAD // MB,),
        in_specs=[
            pl.BlockSpec((NF, MB, F), lambda m: (0, m, 0)),
            pl.BlockSpec((MB, 1), lambda m: (m, 0)),
            pl.BlockSpec((1, D), lambda m: (0, 0)),
            pl.BlockSpec((D, D), lambda m: (0, 0)),
        ],
        out_specs=pl.BlockSpec((NF, MB, F), lambda m: (0, m, 0)),
        out_shape=jax.ShapeDtypeStruct((NF, N_PAD, F), jnp.float32),
    )(s1, dinv_col, b1_row, w2)


def _head_body(ss_ref, st_ref, ds_ref, dt_ref, b2_ref, fc1w_ref, fc1b_ref,
               fc2w_ref, fc2b_ref, out_ref):
    sraw = jnp.concatenate([ss_ref[k] for k in range(NF)], axis=1)
    traw = jnp.concatenate([st_ref[k] for k in range(NF)], axis=1)
    hs = ds_ref[...] * sraw + b2_ref[...]
    ht = dt_ref[...] * traw + b2_ref[...]
    z = hs * ht
    t = jax.nn.relu(
        jnp.dot(z.astype(jnp.bfloat16), fc1w_ref[...].astype(jnp.bfloat16),
                preferred_element_type=jnp.float32)
        + fc1b_ref[...])
    logits = jnp.sum(t * fc2w_ref[...], axis=1, keepdims=True) + fc2b_ref[...]
    out_ref[...] = jax.nn.sigmoid(logits)


def _tc_head(ss, st, dsv, dtv, b2_row, fc1_w, fc1_b_row, fc2_w_row, fc2_b):
    return pl.pallas_call(
        _head_body,
        grid=(B // MB,),
        in_specs=[
            pl.BlockSpec((NF, MB, F), lambda m: (0, m, 0)),
            pl.BlockSpec((NF, MB, F), lambda m: (0, m, 0)),
            pl.BlockSpec((MB, 1), lambda m: (m, 0)),
            pl.BlockSpec((MB, 1), lambda m: (m, 0)),
            pl.BlockSpec((1, D), lambda m: (0, 0)),
            pl.BlockSpec((D, H1), lambda m: (0, 0)),
            pl.BlockSpec((1, H1), lambda m: (0, 0)),
            pl.BlockSpec((1, H1), lambda m: (0, 0)),
            pl.BlockSpec((1, 1), lambda m: (0, 0)),
        ],
        out_specs=pl.BlockSpec((MB, 1), lambda m: (m, 0)),
        out_shape=jax.ShapeDtypeStruct((B, 1), jnp.float32),
    )(ss, st, dsv, dtv, b2_row, fc1_w, fc1_b_row, fc2_w_row, fc2_b)


def kernel(x, edge_index, source_node, target_node,
           W1, b1, W2, b2, fc1_w, fc1_b, fc2_w, fc2_b):
    x_pad = jnp.pad(x, ((0, N_PAD - N), (0, 0)))
    ept = E // NS
    pad_t = NBT * EB - ept
    fill = (N + (jnp.arange(NS * pad_t, dtype=jnp.int32) % (N_PAD - N))
            ).reshape(NS, pad_t)

    def _edges3d(v):
        return jnp.concatenate(
            [v.reshape(NS, ept), fill], axis=1).reshape(NS, NBT, EB)

    src3d = _edges3d(edge_index[0])
    dst3d = _edges3d(edge_index[1])
    sn3d = source_node.reshape(NS, NBP, EB)
    tn3d = target_node.reshape(NS, NBP, EB)

    deg = _sc_degree(dst3d)
    xs1, dinv_col = _tc_matmul_scale(x_pad, W1, deg.reshape(N_PAD, 1))
    dinv = dinv_col.reshape(N_PAD)
    s1 = _sc_aggregate(xs1, src3d, dst3d)
    xs2 = _tc_layer2(s1, dinv_col, b1.reshape(1, D), W2)
    ss, st, dsv, dtv = _sc_agg_link(xs2, src3d, dst3d, dinv, sn3d, tn3d)
    out = _tc_head(ss, st, dsv.reshape(B, 1), dtv.reshape(B, 1),
                   b2.reshape(1, D), fc1_w, fc1_b.reshape(1, H1),
                   fc2_w.reshape(1, H1), fc2_b.reshape(1, 1))
    return out.reshape(B)


# speedup vs baseline: 11.7984x; 1.0044x over previous
import jax
import jax.numpy as jnp
from jax import lax
from jax.experimental import pallas as pl
from jax.experimental.pallas import tpu as pltpu
from jax.experimental.pallas import tpu_sc as plsc

N, E, B, D, H1 = 10000, 100000, 8192, 768, 128
NC, NS, L = 2, 16, 16
N_PAD = 10240
NF = 6
F = D // NF
EB = 128
NBT = 50
E_PAD = NS * NBT * EB
RPT = N_PAD // NS
MB = 512

_MESH = plsc.VectorSubcoreMesh(
    core_axis_name="c", subcore_axis_name="s", num_cores=NC, num_subcores=NS)


def _deg_body(dst3d, deg_hbm, dst_v, ones_v, deg_v, hist_sh):
    cid = lax.axis_index("c")
    sid = lax.axis_index("s")
    for i in range(RPT // L):
        ones_v[pl.ds(L * i, L)] = jnp.full((L,), 1.0, jnp.float32)
    pltpu.sync_copy(ones_v, hist_sh.at[pl.ds(sid * RPT, RPT)])
    pltpu.sync_copy(dst3d.at[sid], dst_v)
    plsc.subcore_barrier()

    @pl.loop(0, NBT)
    def _(b):
        pltpu.sync_copy(ones_v.at[pl.ds(0, EB)],
                        hist_sh.at[dst_v.at[b]], add=True)

    plsc.subcore_barrier()
    @pl.when(cid == 0)
    def _():
        pltpu.sync_copy(hist_sh.at[pl.ds(sid * RPT, RPT)], deg_v)
        pltpu.sync_copy(deg_v, deg_hbm.at[pl.ds(sid * RPT, RPT)])


def _sc_degree(dst3d):
    return pl.kernel(
        _deg_body,
        out_type=jax.ShapeDtypeStruct((N_PAD,), jnp.float32),
        mesh=_MESH,
        scratch_types=[
            pltpu.VMEM((NBT, EB), jnp.int32),
            pltpu.VMEM((RPT,), jnp.float32),
            pltpu.VMEM((RPT,), jnp.float32),
            pltpu.VMEM_SHARED((N_PAD,), jnp.float32),
        ],
    )(dst3d)


NGRP = NBT // 2


def _agg_body(xs_hbm, src3d, dst3d, out_hbm,
              src_v, dst_v, bufs, gsems, ssems, chunk):
    cid = lax.axis_index("c")
    sid = lax.axis_index("s")
    pltpu.sync_copy(src3d.at[sid], src_v)
    pltpu.sync_copy(dst3d.at[sid], dst_v)
    for kk in range(NF // NC):
        k = cid * (NF // NC) + kk

        def _gath(i, j):
            return pltpu.async_copy(xs_hbm.at[k].at[src_v.at[i]],
                                    bufs[j], gsems[j])

        def _gath_wait(i, j):
            pltpu.make_async_copy(xs_hbm.at[k].at[src_v.at[i]],
                                  bufs[j], gsems[j]).wait()

        def _scat(i, j):
            return pltpu.async_copy(bufs[j], chunk.at[dst_v.at[i]],
                                    ssems[j], add=True)

        def _scat_wait(i, j):
            pltpu.make_async_copy(bufs[j], chunk.at[dst_v.at[i]],
                                  ssems[j]).wait()

        _gath(0, 0)
        r0 = sid * RPT
        pltpu.sync_copy(xs_hbm.at[k, pl.ds(r0, RPT)],
                        chunk.at[pl.ds(r0, RPT)])
        plsc.subcore_barrier()

        @pl.loop(0, NGRP)
        def _(g):
            for j in range(2):
                i = 2 * g + j
                jn = (j + 1) % 2
                if j == 0:
                    @pl.when(g > 0)
                    def _():
                        _scat_wait(i - 1, jn)
                    _gath(i + 1, jn)
                else:
                    _scat_wait(i - 1, jn)

                    @pl.when(g < NGRP - 1)
                    def _():
                        _gath(i + 1, jn)
                _gath_wait(i, j)
                _scat(i, j)

        _scat_wait(NBT - 1, 1)
        plsc.subcore_barrier()
        pltpu.sync_copy(chunk.at[pl.ds(r0, RPT)],
                        out_hbm.at[k, pl.ds(r0, RPT)])
        plsc.subcore_barrier()


def _sc_aggregate(xs, src3d, dst3d):
    return pl.kernel(
        _agg_body,
        out_type=jax.ShapeDtypeStruct((NF, N_PAD, F), jnp.float32),
        mesh=_MESH,
        scratch_types=[
            pltpu.VMEM((NBT, EB), jnp.int32),
            pltpu.VMEM((NBT, EB), jnp.int32),
            [pltpu.VMEM((EB, F), jnp.float32) for _ in range(2)],
            [pltpu.SemaphoreType.DMA for _ in range(2)],
            [pltpu.SemaphoreType.DMA for _ in range(2)],
            pltpu.VMEM_SHARED((N_PAD, F), jnp.float32),
        ],
    )(xs, src3d, dst3d)


NBP = B // NS // EB


def _agg_link_body(xs_hbm, src3d, dst3d, dinv_hbm, sn3d, tn3d,
                   ss, st, dsv, dtv,
                   src_v, dst_v, bufs, gsems, ssems, idx_v, dval_v, chunk):
    cid = lax.axis_index("c")
    sid = lax.axis_index("s")
    pltpu.sync_copy(src3d.at[sid], src_v)
    pltpu.sync_copy(dst3d.at[sid], dst_v)
    pltpu.sync_copy(sn3d.at[sid], idx_v.at[pl.ds(0, NBP)])
    pltpu.sync_copy(tn3d.at[sid], idx_v.at[pl.ds(NBP, NBP)])

    @pl.when(cid == 0)
    def _():
        for b in range(NBP):
            pltpu.sync_copy(dinv_hbm.at[idx_v.at[b]], dval_v)
            pltpu.sync_copy(dval_v, dsv.at[pl.ds(sid * NBP * EB + b * EB,
                                                 EB)])

    @pl.when(cid == 1)
    def _():
        for b in range(NBP):
            pltpu.sync_copy(dinv_hbm.at[idx_v.at[NBP + b]], dval_v)
            pltpu.sync_copy(dval_v, dtv.at[pl.ds(sid * NBP * EB + b * EB,
                                                 EB)])

    for kk in range(NF // NC):
        k = cid * (NF // NC) + kk

        def _gath(i, j):
            return pltpu.async_copy(xs_hbm.at[k].at[src_v.at[i]],
                                    bufs[j], gsems[j])

        def _gath_wait(i, j):
            pltpu.make_async_copy(xs_hbm.at[k].at[src_v.at[i]],
                                  bufs[j], gsems[j]).wait()

        def _scat(i, j):
            return pltpu.async_copy(bufs[j], chunk.at[dst_v.at[i]],
                                    ssems[j], add=True)

        def _scat_wait(i, j):
            pltpu.make_async_copy(bufs[j], chunk.at[dst_v.at[i]],
                                  ssems[j]).wait()

        _gath(0, 0)
        r0 = sid * RPT
        pltpu.sync_copy(xs_hbm.at[k, pl.ds(r0, RPT)],
                        chunk.at[pl.ds(r0, RPT)])
        plsc.subcore_barrier()

        @pl.loop(0, NGRP)
        def _(g):
            for j in range(2):
                i = 2 * g + j
                jn = (j + 1) % 2
                if j == 0:
                    @pl.when(g > 0)
                    def _():
                        _scat_wait(i - 1, jn)
                    _gath(i + 1, jn)
                else:
                    _scat_wait(i - 1, jn)

                    @pl.when(g < NGRP - 1)
                    def _():
                        _gath(i + 1, jn)
                _gath_wait(i, j)
                _scat(i, j)

        _scat_wait(NBT - 1, 1)
        plsc.subcore_barrier()
        ub = 0
        for a, rows_out in ((0, ss), (1, st)):
            for b in range(NBP):
                jj = ub % 2
                pltpu.sync_copy(chunk.at[idx_v.at[a * NBP + b]], bufs[jj])
                pltpu.sync_copy(
                    bufs[jj],
                    rows_out.at[k, pl.ds(sid * NBP * EB + b * EB, EB)])
                ub += 1
        plsc.subcore_barrier()


def _sc_agg_link(xs, src3d, dst3d, dinv, sn3d, tn3d):
    return pl.kernel(
        _agg_link_body,
        out_type=(
            jax.ShapeDtypeStruct((NF, B, F), jnp.float32),
            jax.ShapeDtypeStruct((NF, B, F), jnp.float32),
            jax.ShapeDtypeStruct((B,), jnp.float32),
            jax.ShapeDtypeStruct((B,), jnp.float32),
        ),
        mesh=_MESH,
        scratch_types=[
            pltpu.VMEM((NBT, EB), jnp.int32),
            pltpu.VMEM((NBT, EB), jnp.int32),
            [pltpu.VMEM((EB, F), jnp.float32) for _ in range(2)],
            [pltpu.SemaphoreType.DMA for _ in range(2)],
            [pltpu.SemaphoreType.DMA for _ in range(2)],
            pltpu.VMEM((2 * NBP, EB), jnp.int32),
            pltpu.VMEM((EB,), jnp.float32),
            pltpu.VMEM_SHARED((N_PAD, F), jnp.float32),
        ],
    )(xs, src3d, dst3d, dinv, sn3d, tn3d)


def _mm_scale_body(x_ref, w_ref, deg_ref, out_ref, dinv_ref):
    dinv = lax.rsqrt(deg_ref[...])
    dinv_ref[...] = dinv
    acc = jnp.dot(x_ref[...], w_ref[...],
                  preferred_element_type=jnp.float32)
    acc = acc * dinv
    for k in range(NF):
        out_ref[k, :, :] = acc[:, k * F:(k + 1) * F]


def _tc_matmul_scale(x_pad, w, deg_col):
    return pl.pallas_call(
        _mm_scale_body,
        grid=(N_PAD // MB,),
        in_specs=[
            pl.BlockSpec((MB, D), lambda m: (m, 0)),
            pl.BlockSpec((D, D), lambda m: (0, 0)),
            pl.BlockSpec((MB, 1), lambda m: (m, 0)),
        ],
        out_specs=[
            pl.BlockSpec((NF, MB, F), lambda m: (0, m, 0)),
            pl.BlockSpec((MB, 1), lambda m: (m, 0)),
        ],
        out_shape=[
            jax.ShapeDtypeStruct((NF, N_PAD, F), jnp.float32),
            jax.ShapeDtypeStruct((N_PAD, 1), jnp.float32),
        ],
    )(x_pad, w, deg_col)


def _layer2_body(s_ref, dinv_ref, b1_ref, w_ref, out_ref):
    s = jnp.concatenate([s_ref[k] for k in range(NF)], axis=1)
    h = jax.nn.relu(dinv_ref[...] * s + b1_ref[...])
    acc = jnp.dot(h.astype(jnp.bfloat16), w_ref[...],
                  preferred_element_type=jnp.float32)
    acc = acc * dinv_ref[...]
    for k in range(NF):
        out_ref[k, :, :] = acc[:, k * F:(k + 1) * F]


def _tc_layer2(s1, dinv_col, b1_row, w2):
    return pl.pallas_call(
        _layer2_body,
        grid=(N_PAD // MB,),
        in_specs=[
            pl.BlockSpec((NF, MB, F), lambda m: (0, m, 0)),
            pl.BlockSpec((MB, 1), lambda m: (m, 0)),
            pl.BlockSpec((1, D), lambda m: (0, 0)),
            pl.BlockSpec((D, D), lambda m: (0, 0)),
        ],
        out_specs=pl.BlockSpec((NF, MB, F), lambda m: (0, m, 0)),
        out_shape=jax.ShapeDtypeStruct((NF, N_PAD, F), jnp.float32),
    )(s1, dinv_col, b1_row, w2)


def _head_body(ss_ref, st_ref, ds_ref, dt_ref, b2_ref, fc1w_ref, fc1b_ref,
               fc2w_ref, fc2b_ref, out_ref):
    sraw = jnp.concatenate([ss_ref[k] for k in range(NF)], axis=1)
    traw = jnp.concatenate([st_ref[k] for k in range(NF)], axis=1)
    hs = ds_ref[...] * sraw + b2_ref[...]
    ht = dt_ref[...] * traw + b2_ref[...]
    z = hs * ht
    t = jax.nn.relu(
        jnp.dot(z.astype(jnp.bfloat16), fc1w_ref[...],
                preferred_element_type=jnp.float32)
        + fc1b_ref[...])
    logits = jnp.sum(t * fc2w_ref[...], axis=1, keepdims=True) + fc2b_ref[...]
    out_ref[...] = jax.nn.sigmoid(logits)


def _tc_head(ss, st, dsv, dtv, b2_row, fc1_w, fc1_b_row, fc2_w_row, fc2_b):
    return pl.pallas_call(
        _head_body,
        grid=(B // MB,),
        in_specs=[
            pl.BlockSpec((NF, MB, F), lambda m: (0, m, 0)),
            pl.BlockSpec((NF, MB, F), lambda m: (0, m, 0)),
            pl.BlockSpec((MB, 1), lambda m: (m, 0)),
            pl.BlockSpec((MB, 1), lambda m: (m, 0)),
            pl.BlockSpec((1, D), lambda m: (0, 0)),
            pl.BlockSpec((D, H1), lambda m: (0, 0)),
            pl.BlockSpec((1, H1), lambda m: (0, 0)),
            pl.BlockSpec((1, H1), lambda m: (0, 0)),
            pl.BlockSpec((1, 1), lambda m: (0, 0)),
        ],
        out_specs=pl.BlockSpec((MB, 1), lambda m: (m, 0)),
        out_shape=jax.ShapeDtypeStruct((B, 1), jnp.float32),
    )(ss, st, dsv, dtv, b2_row, fc1_w, fc1_b_row, fc2_w_row, fc2_b)


def kernel(x, edge_index, source_node, target_node,
           W1, b1, W2, b2, fc1_w, fc1_b, fc2_w, fc2_b):
    x_pad = jnp.pad(x, ((0, N_PAD - N), (0, 0))).astype(jnp.bfloat16)
    W1 = W1.astype(jnp.bfloat16)
    W2 = W2.astype(jnp.bfloat16)
    fc1_w = fc1_w.astype(jnp.bfloat16)
    ept = E // NS
    pad_t = NBT * EB - ept
    fill = (N + (jnp.arange(NS * pad_t, dtype=jnp.int32) % (N_PAD - N))
            ).reshape(NS, pad_t)

    def _edges3d(v):
        return jnp.concatenate(
            [v.reshape(NS, ept), fill], axis=1).reshape(NS, NBT, EB)

    src3d = _edges3d(edge_index[0])
    dst3d = _edges3d(edge_index[1])
    sn3d = source_node.reshape(NS, NBP, EB)
    tn3d = target_node.reshape(NS, NBP, EB)

    deg = _sc_degree(dst3d)
    xs1, dinv_col = _tc_matmul_scale(x_pad, W1, deg.reshape(N_PAD, 1))
    dinv = dinv_col.reshape(N_PAD)
    s1 = _sc_aggregate(xs1, src3d, dst3d)
    xs2 = _tc_layer2(s1, dinv_col, b1.reshape(1, D), W2)
    ss, st, dsv, dtv = _sc_agg_link(xs2, src3d, dst3d, dinv, sn3d, tn3d)
    out = _tc_head(ss, st, dsv.reshape(B, 1), dtv.reshape(B, 1),
                   b2.reshape(1, D), fc1_w, fc1_b.reshape(1, H1),
                   fc2_w.reshape(1, H1), fc2_b.reshape(1, 1))
    return out.reshape(B)
